# baseline (speedup 1.0000x reference)
import jax
import jax.numpy as jnp
from jax import lax
from jax.experimental import pallas as pl
from jax.experimental.pallas import tpu as pltpu
from jax.experimental.pallas import tpu_sc as plsc

BATCH = 16384
DIM = 64
MARGIN = 1.0
NC = 2
NS = 16
NW = NC * NS
PER_W = BATCH // NW
CHUNK = 64
NCHUNK = PER_W // CHUNK
GROUPS = CHUNK // 16

_EPS = 1e-12


def _rsqrt(x):
    i = lax.bitcast_convert_type(x, jnp.int32)
    i = jnp.int32(0x5F3759DF) - lax.shift_right_arithmetic(i, jnp.int32(1))
    y = lax.bitcast_convert_type(i, jnp.float32)
    for _ in range(3):
        y = y * (1.5 - 0.5 * x * y * y)
    return y


def _body(ph, pt, pr, nh, nt, nr, ent_e, rel_e, ent_t, rel_t, out,
          idx_all,
          ee_b, et_b, re_b, rt_b,
          out_v, sem):
    wid = lax.axis_index("s") * NC + lax.axis_index("c")
    base = wid * PER_W

    lanes = lax.iota(jnp.int32, 16)
    acc = jnp.zeros((16,), jnp.float32)

    for c in range(NCHUNK):
        off = base + c * CHUNK
        for k, src in enumerate((ph, pt, nh, nt, pr, nr)):
            pltpu.sync_copy(src.at[pl.ds(off, CHUNK)],
                            idx_all.at[pl.ds(k * CHUNK, CHUNK)])

        def fetch_ent(i, carry):
            idx = lax.reduce_max(
                plsc.load_gather(idx_all, [jnp.full((16,), i, jnp.int32)]),
                axes=(0,))
            pltpu.async_copy(ent_e.at[pl.ds(idx, 1)],
                             ee_b.at[pl.ds(i, 1)], sem)
            pltpu.async_copy(ent_t.at[pl.ds(idx, 1)],
                             et_b.at[pl.ds(i, 1)], sem)
            return carry

        def fetch_rel(i, carry):
            idx = lax.reduce_max(
                plsc.load_gather(
                    idx_all, [jnp.full((16,), 4 * CHUNK + i, jnp.int32)]),
                axes=(0,))
            pltpu.async_copy(rel_e.at[pl.ds(idx, 1)],
                             re_b.at[pl.ds(i, 1)], sem)
            pltpu.async_copy(rel_t.at[pl.ds(idx, 1)],
                             rt_b.at[pl.ds(i, 1)], sem)
            return carry

        lax.fori_loop(0, 4 * CHUNK, fetch_ent, 0)
        lax.fori_loop(0, 2 * CHUNK, fetch_rel, 0)

        pltpu.make_async_copy(ent_e.at[pl.ds(0, 4 * CHUNK)], ee_b, sem).wait()
        pltpu.make_async_copy(ent_t.at[pl.ds(0, 4 * CHUNK)], et_b, sem).wait()
        pltpu.make_async_copy(rel_e.at[pl.ds(0, 2 * CHUNK)], re_b, sem).wait()
        pltpu.make_async_copy(rel_t.at[pl.ds(0, 2 * CHUNK)], rt_b, sem).wait()

        def group_body(g, acc):
            rh = g * 16 + lanes
            rt_ = CHUNK + g * 16 + lanes
            rh2 = 2 * CHUNK + g * 16 + lanes
            rt2 = 3 * CHUNK + g * 16 + lanes
            rr = g * 16 + lanes
            rr2 = CHUNK + g * 16 + lanes

            def dots(d, carry):
                col = jnp.full((16,), d, jnp.int32)
                he = plsc.load_gather(ee_b, [rh, col])
                ht = plsc.load_gather(et_b, [rh, col])
                te = plsc.load_gather(ee_b, [rt_, col])
                tt = plsc.load_gather(et_b, [rt_, col])
                rt = plsc.load_gather(rt_b, [rr, col])
                hen = plsc.load_gather(ee_b, [rh2, col])
                htn = plsc.load_gather(et_b, [rh2, col])
                ten = plsc.load_gather(ee_b, [rt2, col])
                ttn = plsc.load_gather(et_b, [rt2, col])
                rtn = plsc.load_gather(rt_b, [rr2, col])
                (sh, st, shh, stt, srr, shr, str_,
                 sh2, st2, shh2, stt2, srr2, shr2, str2) = carry
                return (sh + he * ht, st + te * tt,
                        shh + he * he, stt + te * te, srr + rt * rt,
                        shr + he * rt, str_ + te * rt,
                        sh2 + hen * htn, st2 + ten * ttn,
                        shh2 + hen * hen, stt2 + ten * ten,
                        srr2 + rtn * rtn, shr2 + hen * rtn,
                        str2 + ten * rtn)

            z = jnp.zeros((16,), jnp.float32)
            (sh, st, shh, stt, srr, shr, str_,
             sh2, st2, shh2, stt2, srr2, shr2, str2) = lax.fori_loop(
                0, DIM, dots, (z,) * 14)

            nh_sq = shh + 2.0 * sh * shr + sh * sh * srr
            nt_sq = stt + 2.0 * st * str_ + st * st * srr
            nh_sq2 = shh2 + 2.0 * sh2 * shr2 + sh2 * sh2 * srr2
            nt_sq2 = stt2 + 2.0 * st2 * str2 + st2 * st2 * srr2
            ah = _rsqrt(jnp.maximum(nh_sq, _EPS))
            at = _rsqrt(jnp.maximum(nt_sq, _EPS))
            ah2 = _rsqrt(jnp.maximum(nh_sq2, _EPS))
            at2 = _rsqrt(jnp.maximum(nt_sq2, _EPS))
            crt = ah * sh - at * st
            crt2 = ah2 * sh2 - at2 * st2

            def dist(d, carry):
                pacc, nacc = carry
                col = jnp.full((16,), d, jnp.int32)
                he = plsc.load_gather(ee_b, [rh, col])
                te = plsc.load_gather(ee_b, [rt_, col])
                rt = plsc.load_gather(rt_b, [rr, col])
                re = plsc.load_gather(re_b, [rr, col])
                hen = plsc.load_gather(ee_b, [rh2, col])
                ten = plsc.load_gather(ee_b, [rt2, col])
                rtn = plsc.load_gather(rt_b, [rr2, col])
                ren = plsc.load_gather(re_b, [rr2, col])
                p = ah * he - at * te + crt * rt + re
                n = ah2 * hen - at2 * ten + crt2 * rtn + ren
                return pacc + jnp.abs(p), nacc + jnp.abs(n)

            pos, neg = lax.fori_loop(0, DIM, dist, (z, z))
            return acc + jnp.maximum(pos - neg + MARGIN, 0.0)

        acc = lax.fori_loop(0, GROUPS, group_body, acc)

    out_v[...] = acc
    pltpu.sync_copy(out_v, out.at[pl.ds(wid * 16, 16)])


def kernel(x, ent_embeddings, rel_embeddings, ent_transfer, rel_transfer):
    cols = tuple(x[:, j] for j in range(6))
    mesh = plsc.VectorSubcoreMesh(core_axis_name="c", subcore_axis_name="s")
    partials = pl.kernel(
        _body,
        out_type=jax.ShapeDtypeStruct((NW * 16,), jnp.float32),
        mesh=mesh,
        scratch_types=[
            pltpu.VMEM((6 * CHUNK,), jnp.int32),
            pltpu.VMEM((4 * CHUNK, DIM), jnp.float32),
            pltpu.VMEM((4 * CHUNK, DIM), jnp.float32),
            pltpu.VMEM((2 * CHUNK, DIM), jnp.float32),
            pltpu.VMEM((2 * CHUNK, DIM), jnp.float32),
            pltpu.VMEM((16,), jnp.float32),
            pltpu.SemaphoreType.DMA,
        ],
        compiler_params=pltpu.CompilerParams(
            needs_layout_passes=False, use_tc_tiling_on_sc=True),
    )(*cols, ent_embeddings, rel_embeddings, ent_transfer, rel_transfer)
    return jnp.sum(partials)


# speedup vs baseline: 1.0366x; 1.0366x over previous
import jax
import jax.numpy as jnp
from jax import lax
from jax.experimental import pallas as pl
from jax.experimental.pallas import tpu as pltpu
from jax.experimental.pallas import tpu_sc as plsc

BATCH = 16384
DIM = 64
MARGIN = 1.0
NC = 2
NS = 16
NW = NC * NS
PER_W = BATCH // NW
CHUNK = 64
NCHUNK = PER_W // CHUNK
GROUPS = CHUNK // 16

_EPS = 1e-12


def _rsqrt(x):
    i = lax.bitcast_convert_type(x, jnp.int32)
    i = jnp.int32(0x5F3759DF) - lax.shift_right_arithmetic(i, jnp.int32(1))
    y = lax.bitcast_convert_type(i, jnp.float32)
    for _ in range(3):
        y = y * (1.5 - 0.5 * x * y * y)
    return y


def _body(ph, pt, pr, nh, nt, nr, ent_e, rel_e, ent_t, rel_t, out,
          idx_all,
          ee_b, et_b, re_b, rt_b,
          out_v, sem):
    wid = lax.axis_index("s") * NC + lax.axis_index("c")
    base = wid * PER_W

    lanes = lax.iota(jnp.int32, 16)
    acc = jnp.zeros((16,), jnp.float32)

    for c in range(NCHUNK):
        off = base + c * CHUNK
        for k, src in enumerate((ph, pt, nh, nt, pr, nr)):
            pltpu.sync_copy(src.at[pl.ds(off, CHUNK)],
                            idx_all.at[pl.ds(k * CHUNK, CHUNK)])

        def fetch_ent(i, carry):
            idx = lax.reduce_max(
                plsc.load_gather(idx_all, [jnp.full((16,), i, jnp.int32)]),
                axes=(0,))
            pltpu.async_copy(ent_e.at[pl.ds(idx, 1)],
                             ee_b.at[pl.ds(i, 1)], sem)
            pltpu.async_copy(ent_t.at[pl.ds(idx, 1)],
                             et_b.at[pl.ds(i, 1)], sem)
            return carry

        def fetch_rel(i, carry):
            idx = lax.reduce_max(
                plsc.load_gather(
                    idx_all, [jnp.full((16,), 4 * CHUNK + i, jnp.int32)]),
                axes=(0,))
            pltpu.async_copy(rel_e.at[pl.ds(idx, 1)],
                             re_b.at[pl.ds(i, 1)], sem)
            pltpu.async_copy(rel_t.at[pl.ds(idx, 1)],
                             rt_b.at[pl.ds(i, 1)], sem)
            return carry

        lax.fori_loop(0, 4 * CHUNK, fetch_ent, 0)
        lax.fori_loop(0, 2 * CHUNK, fetch_rel, 0)

        pltpu.make_async_copy(ent_e.at[pl.ds(0, 4 * CHUNK)], ee_b, sem).wait()
        pltpu.make_async_copy(ent_t.at[pl.ds(0, 4 * CHUNK)], et_b, sem).wait()
        pltpu.make_async_copy(rel_e.at[pl.ds(0, 2 * CHUNK)], re_b, sem).wait()
        pltpu.make_async_copy(rel_t.at[pl.ds(0, 2 * CHUNK)], rt_b, sem).wait()

        def group_body(g, acc):
            rh = g * 16 + lanes
            rt_ = CHUNK + g * 16 + lanes
            rh2 = 2 * CHUNK + g * 16 + lanes
            rt2 = 3 * CHUNK + g * 16 + lanes
            rr = g * 16 + lanes
            rr2 = CHUNK + g * 16 + lanes

            def dots(k, carry):
                (sh, st, shh, stt, srr, shr, str_,
                 sh2, st2, shh2, stt2, srr2, shr2, str2) = carry
                for j in range(4):
                    col = jnp.full((16,), 4 * k + j, jnp.int32)
                    he = plsc.load_gather(ee_b, [rh, col])
                    ht = plsc.load_gather(et_b, [rh, col])
                    te = plsc.load_gather(ee_b, [rt_, col])
                    tt = plsc.load_gather(et_b, [rt_, col])
                    rt = plsc.load_gather(rt_b, [rr, col])
                    hen = plsc.load_gather(ee_b, [rh2, col])
                    htn = plsc.load_gather(et_b, [rh2, col])
                    ten = plsc.load_gather(ee_b, [rt2, col])
                    ttn = plsc.load_gather(et_b, [rt2, col])
                    rtn = plsc.load_gather(rt_b, [rr2, col])
                    sh = sh + he * ht
                    st = st + te * tt
                    shh = shh + he * he
                    stt = stt + te * te
                    srr = srr + rt * rt
                    shr = shr + he * rt
                    str_ = str_ + te * rt
                    sh2 = sh2 + hen * htn
                    st2 = st2 + ten * ttn
                    shh2 = shh2 + hen * hen
                    stt2 = stt2 + ten * ten
                    srr2 = srr2 + rtn * rtn
                    shr2 = shr2 + hen * rtn
                    str2 = str2 + ten * rtn
                return (sh, st, shh, stt, srr, shr, str_,
                        sh2, st2, shh2, stt2, srr2, shr2, str2)

            z = jnp.zeros((16,), jnp.float32)
            (sh, st, shh, stt, srr, shr, str_,
             sh2, st2, shh2, stt2, srr2, shr2, str2) = lax.fori_loop(
                0, DIM // 4, dots, (z,) * 14)

            nh_sq = shh + 2.0 * sh * shr + sh * sh * srr
            nt_sq = stt + 2.0 * st * str_ + st * st * srr
            nh_sq2 = shh2 + 2.0 * sh2 * shr2 + sh2 * sh2 * srr2
            nt_sq2 = stt2 + 2.0 * st2 * str2 + st2 * st2 * srr2
            ah = _rsqrt(jnp.maximum(nh_sq, _EPS))
            at = _rsqrt(jnp.maximum(nt_sq, _EPS))
            ah2 = _rsqrt(jnp.maximum(nh_sq2, _EPS))
            at2 = _rsqrt(jnp.maximum(nt_sq2, _EPS))
            crt = ah * sh - at * st
            crt2 = ah2 * sh2 - at2 * st2

            def dist(k, carry):
                pacc, nacc = carry
                for j in range(4):
                    col = jnp.full((16,), 4 * k + j, jnp.int32)
                    he = plsc.load_gather(ee_b, [rh, col])
                    te = plsc.load_gather(ee_b, [rt_, col])
                    rt = plsc.load_gather(rt_b, [rr, col])
                    re = plsc.load_gather(re_b, [rr, col])
                    hen = plsc.load_gather(ee_b, [rh2, col])
                    ten = plsc.load_gather(ee_b, [rt2, col])
                    rtn = plsc.load_gather(rt_b, [rr2, col])
                    ren = plsc.load_gather(re_b, [rr2, col])
                    p = ah * he - at * te + crt * rt + re
                    n = ah2 * hen - at2 * ten + crt2 * rtn + ren
                    pacc = pacc + jnp.abs(p)
                    nacc = nacc + jnp.abs(n)
                return pacc, nacc

            pos, neg = lax.fori_loop(0, DIM // 4, dist, (z, z))
            return acc + jnp.maximum(pos - neg + MARGIN, 0.0)

        acc = lax.fori_loop(0, GROUPS, group_body, acc)

    out_v[...] = acc
    pltpu.sync_copy(out_v, out.at[pl.ds(wid * 16, 16)])


def kernel(x, ent_embeddings, rel_embeddings, ent_transfer, rel_transfer):
    cols = tuple(x[:, j] for j in range(6))
    mesh = plsc.VectorSubcoreMesh(core_axis_name="c", subcore_axis_name="s")
    partials = pl.kernel(
        _body,
        out_type=jax.ShapeDtypeStruct((NW * 16,), jnp.float32),
        mesh=mesh,
        scratch_types=[
            pltpu.VMEM((6 * CHUNK,), jnp.int32),
            pltpu.VMEM((4 * CHUNK, DIM), jnp.float32),
            pltpu.VMEM((4 * CHUNK, DIM), jnp.float32),
            pltpu.VMEM((2 * CHUNK, DIM), jnp.float32),
            pltpu.VMEM((2 * CHUNK, DIM), jnp.float32),
            pltpu.VMEM((16,), jnp.float32),
            pltpu.SemaphoreType.DMA,
        ],
        compiler_params=pltpu.CompilerParams(
            needs_layout_passes=False, use_tc_tiling_on_sc=True),
    )(*cols, ent_embeddings, rel_embeddings, ent_transfer, rel_transfer)
    return jnp.sum(partials)
